# baseline (speedup 1.0000x reference)
import functools

import jax
import jax.numpy as jnp
from jax import lax
from jax.experimental import pallas as pl
from jax.experimental.pallas import tpu as pltpu
from jax.experimental.pallas import tpu_sc as plsc

NUM_ROWS = 100000
N_AGENTS = 8
N_ACTIONS = 16
BATCH = 16384
D = N_AGENTS * N_ACTIONS
NB = 16

NC = 2
NS = 16
NW = NC * NS

BKW = 3200
NT = BKW // 128
C0_MAX = 96768
TAIL0 = 99968
MAGIC = 2622
SEG = 384
SCH = 128
PAD_ROW = BATCH

_mesh = plsc.VectorSubcoreMesh(core_axis_name="c", subcore_axis_name="s")


@functools.partial(
    pl.kernel,
    mesh=_mesh,
    out_type=jax.ShapeDtypeStruct((BATCH + SCH, D), jnp.float32),
    compiler_params=pltpu.CompilerParams(needs_layout_passes=False),
    scratch_types=[
        pltpu.VMEM((BATCH,), jnp.int32),
        pltpu.VMEM((SEG // SCH, SCH), jnp.int32),
        pltpu.VMEM((SEG,), jnp.int32),
        pltpu.VMEM((8, BKW), jnp.float32),
        pltpu.VMEM((8, BKW), jnp.float32),
        pltpu.VMEM((D * 32,), jnp.float32),
        pltpu.VMEM((SEG, D), jnp.float32),
        pltpu.SemaphoreType.DMA,
        pltpu.SemaphoreType.DMA,
    ],
)
def _kern(table_hbm, tail_hbm, idx_hbm, out_hbm,
          idx_v, pos2_v, col_v, chunk_a, chunk_b, tail_f, rowbuf_v,
          sem_l, sem_w):
    wid = lax.axis_index("s") * NC + lax.axis_index("c")
    m = wid
    c0 = jnp.minimum(m * BKW, C0_MAX)
    iota16 = lax.iota(jnp.int32, 16)
    pltpu.sync_copy(idx_hbm, idx_v)
    pltpu.sync_copy(tail_hbm, tail_f)

    padpos = jnp.full((16,), PAD_ROW, jnp.int32)
    padcol = jnp.broadcast_to(c0, (16,)).astype(jnp.int32)

    def do_seg(sg):
        lo = sg * SEG

        @plsc.parallel_loop(0, SEG // 16, unroll=8)
        def _fill(i):
            pos2_v[(i * 16) // SCH, pl.ds((i * 16) % SCH, 16)] = padpos
            col_v[pl.ds(i * 16, 16)] = padcol

        @plsc.parallel_loop(0, BATCH // 16, unroll=4,
                            carry=jnp.zeros((16,), jnp.int32))
        def _collect(i, wptr):
            col = idx_v[pl.ds(i * 16, 16)]
            q = ((col >> 7) * MAGIC) >> 16
            mask = q == m
            cs = plsc.cumsum(jnp.where(mask, 1, 0))
            wpos = wptr + cs - 1
            wl = wpos - lo
            mw = mask & (wpos >= lo) & (wl < SEG)
            plsc.store_scatter(pos2_v, [wl >> 7, wl & (SCH - 1)],
                               iota16 + i * 16, mask=mw)
            plsc.store_scatter(col_v, [wl], col, mask=mw)
            return wptr + plsc.all_reduce_population_count(mask)

        bufs = [chunk_a, chunk_b]

        def issue(t):
            return pltpu.async_copy(
                table_hbm.at[pl.ds(t * 8, 8), pl.ds(c0, BKW)],
                bufs[t % 2], sem_l)

        h = issue(0)
        for t in range(NB):
            chunk_v = bufs[t % 2]
            h.wait()
            if t < NB - 1:
                h = issue(t + 1)
            t8 = t * 8

            @plsc.parallel_loop(0, SEG // 16, unroll=2)
            def _g(i, chunk_v=chunk_v, t8=t8):
                col = col_v[pl.ds(i * 16, 16)]
                cp = jnp.minimum(col - c0, BKW - 1)
                rowv = iota16 + i * 16
                for s in range(8):
                    svec = jnp.full((16,), s, jnp.int32)
                    pcol = jnp.broadcast_to(t8 + s, (16,)).astype(jnp.int32)
                    v = plsc.load_gather(chunk_v, [svec, cp])
                    plsc.store_scatter(rowbuf_v, [rowv, pcol], v)

        @plsc.parallel_loop(0, SEG // 16)
        def _tailfix(i):
            col = col_v[pl.ds(i * 16, 16)]
            in_tail = col >= TAIL0
            t_any = jnp.max(jnp.where(in_tail, 1, 0))

            @pl.when(t_any > 0)
            def _():
                toff = col - TAIL0
                rowv = iota16 + i * 16
                for p in range(D):
                    pvec = jnp.full((16,), p, jnp.int32)
                    tv = plsc.load_gather(tail_f, [toff + p * 32],
                                          mask=in_tail)
                    plsc.store_scatter(rowbuf_v, [rowv, pvec], tv,
                                       mask=in_tail)

        writes = [
            pltpu.async_copy(
                rowbuf_v.at[pl.ds(k * SCH, SCH)],
                out_hbm.at[pos2_v.at[k]],
                sem_w)
            for k in range(SEG // SCH)
        ]
        for cpy in writes:
            cpy.wait()
        return jnp.max(_collect)

    n = do_seg(0)
    nseg = (n + SEG - 1) // SEG

    def seg_rest(sg, _):
        do_seg(sg)
        return 0

    lax.fori_loop(1, nseg, seg_rest, 0)


def kernel(x, params):
    table = params.transpose(1, 2, 0).reshape(D, NUM_ROWS)
    tail = params[TAIL0:].transpose(1, 2, 0).reshape(D * 32)
    idx = x.reshape(BATCH).astype(jnp.int32)
    out2 = _kern(table, tail, idx)
    return out2[:BATCH].reshape(BATCH, N_AGENTS, N_ACTIONS)


# speedup vs baseline: 5.3302x; 5.3302x over previous
import functools

import jax
import jax.numpy as jnp
from jax import lax
from jax.experimental import pallas as pl
from jax.experimental.pallas import tpu as pltpu
from jax.experimental.pallas import tpu_sc as plsc

NUM_ROWS = 100000
N_AGENTS = 8
N_ACTIONS = 16
BATCH = 16384
D = N_AGENTS * N_ACTIONS

NC = 2
NS = 16
NW = NC * NS
B_PER_W = BATCH // NW
CHUNK = 128
N_CHUNKS = B_PER_W // CHUNK

_mesh = plsc.VectorSubcoreMesh(core_axis_name="c", subcore_axis_name="s")


@functools.partial(
    pl.kernel,
    mesh=_mesh,
    out_type=jax.ShapeDtypeStruct((BATCH, D), jnp.float32),
    scratch_types=[
        pltpu.VMEM((N_CHUNKS, CHUNK), jnp.int32),
        pltpu.VMEM((B_PER_W, D), jnp.float32),
        pltpu.SemaphoreType.DMA,
    ],
)
def _gather(table_hbm, idx_hbm, out_hbm, idx_v, rows_v, sem):
    wid = lax.axis_index("s") * NC + lax.axis_index("c")
    base = wid * N_CHUNKS
    pltpu.sync_copy(idx_hbm.at[pl.ds(base, N_CHUNKS)], idx_v)
    copies = [
        pltpu.async_copy(
            table_hbm.at[idx_v.at[j]],
            rows_v.at[pl.ds(j * CHUNK, CHUNK)],
            sem,
        )
        for j in range(N_CHUNKS)
    ]
    for c in copies:
        c.wait()
    pltpu.sync_copy(rows_v, out_hbm.at[pl.ds(base * CHUNK, B_PER_W)])


def kernel(x, params):
    table = params.reshape(NUM_ROWS, D)
    idx = x.reshape(BATCH // CHUNK, CHUNK).astype(jnp.int32)
    out = _gather(table, idx)
    return out.reshape(BATCH, N_AGENTS, N_ACTIONS)
